# Initial kernel scaffold; baseline (speedup 1.0000x reference)
import jax
import jax.numpy as jnp
from jax.experimental import pallas as pl


def kernel(input_ids, embed_table):
    raise NotImplementedError("write your pallas kernel here")


# speedup vs baseline: 1.6290x; 1.6290x over previous
import functools

import jax
import jax.numpy as jnp
from jax import lax
from jax.experimental import pallas as pl
from jax.experimental.pallas import tpu as pltpu
from jax.experimental.pallas import tpu_sc as plsc

VOCAB = 65024
DIM = 1024
BATCH = 4
SEQ = 8192

_INFO = plsc.get_sparse_core_info()
_NC = _INFO.num_cores
_NS = _INFO.num_subcores
_NW = _NC * _NS
_B = BATCH * SEQ
_BPW = _B // _NW
_CHUNK = 64
_NCHUNK = _BPW // _CHUNK


def _body(idx_hbm, table_hbm, out_hbm, idx_v, rows_v, gsem):
    wid = lax.axis_index("s") * _NC + lax.axis_index("c")
    base = wid * _BPW
    pltpu.sync_copy(idx_hbm.at[pl.ds(base, _BPW)], idx_v)

    def chunk(i, carry):
        off = i * _CHUNK
        pltpu.async_copy(table_hbm.at[idx_v.at[pl.ds(off, _CHUNK)]], rows_v, gsem).wait()
        pltpu.sync_copy(rows_v, out_hbm.at[pl.ds(base + off, _CHUNK)])
        return carry

    lax.fori_loop(0, _NCHUNK, chunk, 0)


@jax.jit
def _embed(ids_flat, table):
    run = functools.partial(
        pl.kernel,
        out_type=jax.ShapeDtypeStruct((_B, DIM), jnp.float32),
        mesh=plsc.VectorSubcoreMesh(core_axis_name="c", subcore_axis_name="s"),
        scratch_types=[
            pltpu.VMEM((_BPW,), jnp.int32),
            pltpu.VMEM((_CHUNK, DIM), jnp.float32),
            pltpu.SemaphoreType.DMA,
        ],
    )(_body)
    return run(ids_flat, table)


def kernel(input_ids, embed_table):
    ids_flat = input_ids.reshape(-1).astype(jnp.int32)
    out = _embed(ids_flat, embed_table)
    return out.reshape(BATCH, SEQ, DIM)


# speedup vs baseline: 1.6728x; 1.0269x over previous
import functools

import jax
import jax.numpy as jnp
from jax import lax
from jax.experimental import pallas as pl
from jax.experimental.pallas import tpu as pltpu
from jax.experimental.pallas import tpu_sc as plsc

VOCAB = 65024
DIM = 1024
BATCH = 4
SEQ = 8192

_INFO = plsc.get_sparse_core_info()
_NC = _INFO.num_cores
_NS = _INFO.num_subcores
_NW = _NC * _NS
_B = BATCH * SEQ
_BPW = _B // _NW
_C = 32
_NBUF = 2
_NCH = _BPW // _C
_OUTER = _NCH // _NBUF


def _body(idx_hbm, table_hbm, out_hbm, idx_v, *rest):
    bufs = rest[:_NBUF]
    gsems = rest[_NBUF:2 * _NBUF]
    osems = rest[2 * _NBUF:3 * _NBUF]

    wid = lax.axis_index("s") * _NC + lax.axis_index("c")
    base = wid * _BPW
    pltpu.sync_copy(idx_hbm.at[pl.ds(base, _BPW)], idx_v)

    def gather_start(ci, b):
        pltpu.async_copy(
            table_hbm.at[idx_v.at[pl.ds(ci * _C, _C)]], bufs[b], gsems[b])

    def gather_wait(ci, b):
        pltpu.make_async_copy(
            table_hbm.at[idx_v.at[pl.ds(ci * _C, _C)]], bufs[b], gsems[b]).wait()

    def write_start(ci, b):
        pltpu.async_copy(bufs[b], out_hbm.at[pl.ds(base + ci * _C, _C)], osems[b])

    def write_wait(ci, b):
        pltpu.make_async_copy(
            bufs[b], out_hbm.at[pl.ds(base + ci * _C, _C)], osems[b]).wait()

    for b in range(_NBUF):
        gather_start(b, b)

    def outer(g, carry):
        c0 = g * _NBUF
        for b in range(_NBUF):
            gather_wait(c0 + b, b)
            write_start(c0 + b, b)
        nxt = c0 + _NBUF

        @pl.when(g + 1 < _OUTER)
        def _():
            for b in range(_NBUF):
                write_wait(c0 + b, b)
                gather_start(nxt + b, b)
        return carry

    lax.fori_loop(0, _OUTER, outer, 0)
    for b in range(_NBUF):
        write_wait((_OUTER - 1) * _NBUF + b, b)


@jax.jit
def _embed(ids_flat, table):
    scratch = [pltpu.VMEM((_BPW,), jnp.int32)]
    scratch += [pltpu.VMEM((_C, DIM), jnp.float32) for _ in range(_NBUF)]
    scratch += [pltpu.SemaphoreType.DMA for _ in range(2 * _NBUF)]
    run = functools.partial(
        pl.kernel,
        out_type=jax.ShapeDtypeStruct((_B, DIM), jnp.float32),
        mesh=plsc.VectorSubcoreMesh(core_axis_name="c", subcore_axis_name="s"),
        scratch_types=scratch,
    )(_body)
    return run(ids_flat, table)


def kernel(input_ids, embed_table):
    ids_flat = input_ids.reshape(-1).astype(jnp.int32)
    out = _embed(ids_flat, embed_table)
    return out.reshape(BATCH, SEQ, DIM)


# speedup vs baseline: 1.6854x; 1.0075x over previous
import functools

import jax
import jax.numpy as jnp
from jax import lax
from jax.experimental import pallas as pl
from jax.experimental.pallas import tpu as pltpu
from jax.experimental.pallas import tpu_sc as plsc

VOCAB = 65024
DIM = 1024
BATCH = 4
SEQ = 8192

_INFO = plsc.get_sparse_core_info()
_NC = _INFO.num_cores
_NS = _INFO.num_subcores
_NW = _NC * _NS
_B = BATCH * SEQ
_BPW = _B // _NW
_WPB = SEQ // _BPW
_C = 32
_NCH = _BPW // _C
_ROUNDS = _NCH // 2


def _body(idx_hbm, table_hbm, out_hbm, idx_v, buf0, buf1, gs0, gs1, ws0, ws1):
    wid = lax.axis_index("s") * _NC + lax.axis_index("c")
    row = wid // _WPB
    col = (wid % _WPB) * _BPW
    bufs = (buf0, buf1)
    gsems = (gs0, gs1)
    wsems = (ws0, ws1)

    pltpu.sync_copy(idx_hbm.at[row, pl.ds(col, _BPW)], idx_v)

    def gather_start(ci, b):
        pltpu.async_copy(
            table_hbm.at[idx_v.at[pl.ds(ci * _C, _C)]], bufs[b], gsems[b])

    def gather_wait(ci, b):
        pltpu.make_async_copy(
            table_hbm.at[idx_v.at[pl.ds(ci * _C, _C)]], bufs[b], gsems[b]).wait()

    def write_start(ci, b):
        pltpu.async_copy(
            bufs[b], out_hbm.at[row, pl.ds(col + ci * _C, _C)], wsems[b])

    def write_wait(ci, b):
        pltpu.make_async_copy(
            bufs[b], out_hbm.at[row, pl.ds(col + ci * _C, _C)], wsems[b]).wait()

    gather_start(0, 0)
    gather_start(1, 1)

    def round_(g, carry):
        i0 = g * 2
        gather_wait(i0, 0)
        write_start(i0, 0)
        gather_wait(i0 + 1, 1)
        write_start(i0 + 1, 1)
        write_wait(i0, 0)
        gather_start(i0 + 2, 0)
        write_wait(i0 + 1, 1)
        gather_start(i0 + 3, 1)
        return carry

    lax.fori_loop(0, _ROUNDS - 1, round_, 0)
    last = _NCH - 2
    gather_wait(last, 0)
    write_start(last, 0)
    gather_wait(last + 1, 1)
    write_start(last + 1, 1)
    write_wait(last, 0)
    write_wait(last + 1, 1)


@jax.jit
def _embed(ids, table):
    run = functools.partial(
        pl.kernel,
        out_type=jax.ShapeDtypeStruct((BATCH, SEQ, DIM), jnp.float32),
        mesh=plsc.VectorSubcoreMesh(core_axis_name="c", subcore_axis_name="s"),
        scratch_types=[
            pltpu.VMEM((_BPW,), jnp.int32),
            pltpu.VMEM((_C, DIM), jnp.float32),
            pltpu.VMEM((_C, DIM), jnp.float32),
            pltpu.SemaphoreType.DMA,
            pltpu.SemaphoreType.DMA,
            pltpu.SemaphoreType.DMA,
            pltpu.SemaphoreType.DMA,
        ],
    )(_body)
    return run(ids, table)


def kernel(input_ids, embed_table):
    if input_ids.dtype != jnp.int32:
        input_ids = input_ids.astype(jnp.int32)
    return _embed(input_ids, embed_table)
